# baseline (speedup 1.0000x reference)
import functools

import jax
import jax.numpy as jnp
from jax import lax
from jax.experimental import pallas as pl
from jax.experimental.pallas import tpu as pltpu
from jax.experimental.pallas import tpu_sc as plsc

CHUNK = 128
GPC = 2
NBUF = 4


@functools.lru_cache(maxsize=None)
def _make_gather(n_total: int, vocab: int, embed: int):
    info = plsc.get_sparse_core_info()
    nc, ns = info.num_cores, info.num_subcores
    nw = nc * ns
    rows_g = CHUNK * GPC
    assert n_total % (nw * rows_g * NBUF) == 0
    per_w = n_total // nw
    n_groups = per_w // rows_g
    n_iter = n_groups // NBUF
    assert n_iter >= 2
    mesh = plsc.VectorSubcoreMesh(core_axis_name="c", subcore_axis_name="s")

    @functools.partial(
        pl.kernel,
        mesh=mesh,
        out_type=jax.ShapeDtypeStruct((n_total, embed), jnp.float32),
        scratch_types=[
            pltpu.VMEM((per_w,), jnp.int32),
        ]
        + [pltpu.VMEM((rows_g, embed), jnp.float32) for _ in range(NBUF)]
        + [pltpu.SemaphoreType.DMA for _ in range(2 * NBUF)],
        compiler_params=pltpu.CompilerParams(
            use_tc_tiling_on_sc=False,
            disable_bounds_checks=True,
            disable_semaphore_checks=True,
            skip_device_barrier=True,
        ),
    )
    def gather_kernel(idx_hbm, table_hbm, out_hbm, idx_v, *rest):
        bufs = rest[:NBUF]
        gsem = rest[NBUF : 2 * NBUF]
        ssem = rest[2 * NBUF :]
        wid = lax.axis_index("s") * nc + lax.axis_index("c")
        base = wid * per_w
        pltpu.sync_copy(idx_hbm.at[pl.ds(base, per_w)], idx_v)

        def fire_gathers(g, b):
            for j in range(GPC):
                off = g * rows_g + j * CHUNK
                pltpu.async_copy(
                    table_hbm.at[idx_v.at[pl.ds(off, CHUNK)]],
                    bufs[b].at[pl.ds(j * CHUNK, CHUNK)],
                    gsem[b],
                )

        def wait_gathers(b):
            for j in range(GPC):
                pltpu.make_async_copy(
                    out_hbm.at[pl.ds(0, CHUNK)],
                    bufs[b].at[pl.ds(j * CHUNK, CHUNK)],
                    gsem[b],
                ).wait()

        def fire_store(g, b):
            pltpu.async_copy(
                bufs[b], out_hbm.at[pl.ds(base + g * rows_g, rows_g)], ssem[b]
            )

        def wait_store(b):
            pltpu.make_async_copy(
                bufs[b], out_hbm.at[pl.ds(base, rows_g)], ssem[b]
            ).wait()

        def step(g, b, do_wait_store, do_fire_gather):
            wait_gathers(b)
            fire_store(g, b)
            if do_wait_store:
                wait_store((b + 2) % NBUF)
            if do_fire_gather:
                fire_gathers(g + 2, (b + 2) % NBUF)

        fire_gathers(0, 0)
        fire_gathers(1, 1)

        for b in range(NBUF):
            step(b, b, b >= 2, True)

        def body(t, carry):
            g0 = t * NBUF
            for b in range(NBUF):
                step(g0 + b, b, True, True)
            return carry

        lax.fori_loop(1, n_iter - 1, body, 0)

        g0 = (n_iter - 1) * NBUF
        for b in range(NBUF):
            g = g0 + b
            step(g, b, True, g + 2 < n_groups)

        wait_store((NBUF - 2) % NBUF)
        wait_store((NBUF - 1) % NBUF)

    return gather_kernel


def kernel(x, table):
    b, s = x.shape
    vocab, embed = table.shape
    x_flat = x.reshape(-1).astype(jnp.int32)
    out = _make_gather(b * s, vocab, embed)(x_flat, table)
    return out.reshape(b, s, embed)


# speedup vs baseline: 1.0023x; 1.0023x over previous
import functools

import jax
import jax.numpy as jnp
from jax import lax
from jax.experimental import pallas as pl
from jax.experimental.pallas import tpu as pltpu
from jax.experimental.pallas import tpu_sc as plsc

NBUF = 4


@functools.lru_cache(maxsize=None)
def _make_gather(batch: int, seq: int, vocab: int, embed: int):
    info = plsc.get_sparse_core_info()
    nc, ns = info.num_cores, info.num_subcores
    nw = nc * ns
    assert batch % (nw * NBUF) == 0
    rows_w = batch // nw
    n_pass = rows_w // NBUF
    assert n_pass >= 2
    chunks = []
    off = 0
    while off < seq:
        w = min(128, seq - off)
        chunks.append((off, w))
        off += w
    assert all(o % 8 == 0 for o, _ in chunks)
    mesh = plsc.VectorSubcoreMesh(core_axis_name="c", subcore_axis_name="s")

    @functools.partial(
        pl.kernel,
        mesh=mesh,
        out_type=jax.ShapeDtypeStruct((batch, seq, embed), jnp.float32),
        scratch_types=[
            pltpu.VMEM((rows_w, seq), jnp.int32),
        ]
        + [pltpu.VMEM((seq, embed), jnp.float32) for _ in range(NBUF)]
        + [pltpu.SemaphoreType.DMA for _ in range(2 * NBUF)],
        compiler_params=pltpu.CompilerParams(use_tc_tiling_on_sc=False),
    )
    def gather_kernel(idx_hbm, table_hbm, out_hbm, idx_v, *rest):
        bufs = rest[:NBUF]
        gsem = rest[NBUF : 2 * NBUF]
        ssem = rest[2 * NBUF :]
        wid = lax.axis_index("s") * nc + lax.axis_index("c")
        row0 = wid * rows_w
        pltpu.sync_copy(idx_hbm.at[pl.ds(row0, rows_w), :], idx_v)

        def fire_gathers(r, b):
            for o, w in chunks:
                pltpu.async_copy(
                    table_hbm.at[idx_v.at[r, pl.ds(o, w)]],
                    bufs[b].at[pl.ds(o, w)],
                    gsem[b],
                )

        def wait_gathers(b):
            for o, w in chunks:
                pltpu.make_async_copy(
                    out_hbm.at[0, pl.ds(o, w), :],
                    bufs[b].at[pl.ds(o, w)],
                    gsem[b],
                ).wait()

        def fire_store(r, b):
            pltpu.async_copy(bufs[b], out_hbm.at[row0 + r], ssem[b])

        def wait_store(b):
            pltpu.make_async_copy(bufs[b], out_hbm.at[0], ssem[b]).wait()

        def step(r, b, do_wait_store, do_fire_gather):
            wait_gathers(b)
            fire_store(r, b)
            if do_wait_store:
                wait_store((b + 2) % NBUF)
            if do_fire_gather:
                fire_gathers(r + 2, (b + 2) % NBUF)

        fire_gathers(0, 0)
        fire_gathers(1, 1)

        for b in range(NBUF):
            step(b, b, b >= 2, True)

        def body(t, carry):
            r_base = t * NBUF
            for b in range(NBUF):
                step(r_base + b, b, True, True)
            return carry

        lax.fori_loop(1, n_pass - 1, body, 0)

        r_base = (n_pass - 1) * NBUF
        for b in range(NBUF):
            r = r_base + b
            step(r, b, True, r + 2 < rows_w)

        wait_store((NBUF - 2) % NBUF)
        wait_store((NBUF - 1) % NBUF)

    return gather_kernel


def kernel(x, table):
    b, s = x.shape
    vocab, embed = table.shape
    return _make_gather(b, s, vocab, embed)(x.astype(jnp.int32), table)
